# Initial kernel scaffold; baseline (speedup 1.0000x reference)
import jax
import jax.numpy as jnp
from jax.experimental import pallas as pl


def kernel(mailbox_node_h, mailbox_attn, mailbox_edge_h, node_h, node_features, W1, b1, W2, b2, W3, b3):
    raise NotImplementedError("write your pallas kernel here")


# speedup vs baseline: 1.3722x; 1.3722x over previous
import jax
import jax.numpy as jnp
from jax.experimental import pallas as pl
from jax.experimental.pallas import tpu as pltpu

N = 10000
DEG = 16
D = 256
HIDDEN = 512
OUT = 256
BN = 400


def _fused(mnh_ref, attn_ref, meh_ref, nh_ref, nf_ref,
           w1e_ref, w1n_ref, w1h_ref, w1f_ref, b1_ref,
           w2_ref, b2_ref, w3_ref, b3_ref, out_ref):
    attn = attn_ref[...]
    node_msg = jnp.sum(mnh_ref[...] * attn[:, :, None], axis=1)
    edge_msg = jnp.sum(meh_ref[...], axis=1)
    h = (jnp.dot(edge_msg, w1e_ref[...], preferred_element_type=jnp.float32)
         + jnp.dot(node_msg, w1n_ref[...], preferred_element_type=jnp.float32)
         + jnp.dot(nh_ref[...], w1h_ref[...], preferred_element_type=jnp.float32)
         + jnp.dot(nf_ref[...], w1f_ref[...], preferred_element_type=jnp.float32)
         + b1_ref[...])
    h = jnp.maximum(h, 0.0)
    h = jnp.dot(h, w2_ref[...], preferred_element_type=jnp.float32) + b2_ref[...]
    h = jnp.maximum(h, 0.0)
    out_ref[...] = jnp.dot(h, w3_ref[...], preferred_element_type=jnp.float32) + b3_ref[...]


def kernel(mailbox_node_h, mailbox_attn, mailbox_edge_h, node_h, node_features,
           W1, b1, W2, b2, W3, b3):
    attn2d = mailbox_attn[:, :, 0]
    w1e = W1[0 * D:1 * D]
    w1n = W1[1 * D:2 * D]
    w1h = W1[2 * D:3 * D]
    w1f = W1[3 * D:4 * D]
    b1r = b1.reshape(1, HIDDEN)
    b2r = b2.reshape(1, HIDDEN)
    b3r = b3.reshape(1, OUT)

    grid = (N // BN,)
    row = lambda i: (i, 0)
    row3 = lambda i: (i, 0, 0)
    fixed = lambda i: (0, 0)

    return pl.pallas_call(
        _fused,
        grid=grid,
        in_specs=[
            pl.BlockSpec((BN, DEG, D), row3),
            pl.BlockSpec((BN, DEG), row),
            pl.BlockSpec((BN, DEG, D), row3),
            pl.BlockSpec((BN, D), row),
            pl.BlockSpec((BN, D), row),
            pl.BlockSpec((D, HIDDEN), fixed),
            pl.BlockSpec((D, HIDDEN), fixed),
            pl.BlockSpec((D, HIDDEN), fixed),
            pl.BlockSpec((D, HIDDEN), fixed),
            pl.BlockSpec((1, HIDDEN), fixed),
            pl.BlockSpec((HIDDEN, HIDDEN), fixed),
            pl.BlockSpec((1, HIDDEN), fixed),
            pl.BlockSpec((HIDDEN, OUT), fixed),
            pl.BlockSpec((1, OUT), fixed),
        ],
        out_specs=pl.BlockSpec((BN, OUT), row),
        out_shape=jax.ShapeDtypeStruct((N, OUT), jnp.float32),
        compiler_params=pltpu.CompilerParams(
            dimension_semantics=("arbitrary",),
        ),
    )(mailbox_node_h, attn2d, mailbox_edge_h, node_h, node_features,
      w1e, w1n, w1h, w1f, b1r, W2, b2r, W3, b3r)
